# baseline (speedup 1.0000x reference)
import functools

import jax
import jax.numpy as jnp
import numpy as np
from jax import lax
from jax.experimental import pallas as pl
from jax.experimental.pallas import tpu as pltpu
from jax.experimental.pallas import tpu_sc as plsc

VOCAB = 1000000
DIM = 64
BATCH = 1024
SEQ = 200

_INFO = plsc.get_sparse_core_info()
NC, NS = _INFO.num_cores, _INFO.num_subcores
NW = NC * NS
N_TOK = BATCH * SEQ
PER_W = N_TOK // NW
SEQ_PER_W = PER_W // SEQ
CH_SEQ = 2
CH = CH_SEQ * SEQ
NCHUNK = PER_W // CH
NGRP = CH // 16


def _sin_pe(seq_len, dim):
    pos = np.arange(seq_len, dtype=np.float32)[:, None]
    i = np.arange(0, dim, 2, dtype=np.float32)
    div = np.exp(-np.log(10000.0) * i / dim)
    pe = np.zeros((seq_len, dim), dtype=np.float32)
    pe[:, 0::2] = np.sin(pos * div)
    pe[:, 1::2] = np.cos(pos * div)
    return pe


_PE = _sin_pe(SEQ, DIM)

_MESH = plsc.VectorSubcoreMesh(core_axis_name="c", subcore_axis_name="s")


@functools.partial(
    pl.kernel,
    out_type=jax.ShapeDtypeStruct((BATCH, SEQ, DIM), jnp.float32),
    mesh=_MESH,
    scratch_types=[
        pltpu.VMEM((CH,), jnp.int32),
        pltpu.VMEM((CH,), jnp.int32),
        pltpu.VMEM((SEQ, DIM), jnp.float32),
        pltpu.VMEM((CH, DIM), jnp.float32),
        pltpu.VMEM((CH, DIM), jnp.float32),
        pltpu.SemaphoreType.DMA,
        pltpu.SemaphoreType.DMA,
        pltpu.SemaphoreType.DMA,
        pltpu.SemaphoreType.DMA,
    ],
)
def _embed_sc(tok_hbm, table_hbm, pe_hbm, out_hbm,
              idx0, idx1, pe_v, rows0, rows1, gsem0, gsem1, wsem0, wsem1):
    wid = lax.axis_index("s") * NC + lax.axis_index("c")
    base = wid * PER_W
    b0 = wid * SEQ_PER_W
    pltpu.sync_copy(pe_hbm, pe_v)

    idxs = (idx0, idx1)
    bufs = (rows0, rows1)
    gsems = (gsem0, gsem1)
    wsems = (wsem0, wsem1)

    def fire_gather(s, b):
        buf = bufs[b]
        sem = gsems[b]
        idx_v = idxs[b]
        pltpu.sync_copy(tok_hbm.at[pl.ds(base + s * CH, CH)], idx_v)

        def grp(k, c):
            tv = idx_v[pl.ds(k * 16, 16)]
            for j in range(16):
                t = tv[j]
                pltpu.async_copy(table_hbm.at[t], buf.at[k * 16 + j], sem)
            return c

        lax.fori_loop(0, NGRP, grp, 0, unroll=2)

    def wait_gather(b):
        pltpu.make_async_copy(
            table_hbm.at[pl.ds(0, CH)], bufs[b], gsems[b]
        ).wait()

    fire_gather(0, 0)
    writes = []
    for g in range(NCHUNK):
        b = g & 1
        if g + 1 < NCHUNK:
            if g >= 1:
                writes[2 * (g - 1)].wait()
                writes[2 * (g - 1) + 1].wait()
            fire_gather(g + 1, 1 - b)
        wait_gather(b)

        buf = bufs[b]

        @plsc.parallel_loop(0, SEQ, unroll=4)
        def _add(r):
            for k in range(CH_SEQ):
                for d in range(DIM // 16):
                    buf[k * SEQ + r, pl.ds(d * 16, 16)] = (
                        buf[k * SEQ + r, pl.ds(d * 16, 16)]
                        + pe_v[r, pl.ds(d * 16, 16)]
                    )

        for k in range(CH_SEQ):
            writes.append(pltpu.async_copy(
                buf.at[pl.ds(k * SEQ, SEQ)],
                out_hbm.at[b0 + g * CH_SEQ + k],
                wsems[b],
            ))
    for w in writes[-4:]:
        w.wait()


def kernel(tokens, table):
    tok_flat = tokens.astype(jnp.int32).reshape(-1)
    return _embed_sc(tok_flat, table, jnp.asarray(_PE))


# speedup vs baseline: 1.0095x; 1.0095x over previous
import functools

import jax
import jax.numpy as jnp
import numpy as np
from jax import lax
from jax.experimental import pallas as pl
from jax.experimental.pallas import tpu as pltpu
from jax.experimental.pallas import tpu_sc as plsc

VOCAB = 1000000
DIM = 64
BATCH = 1024
SEQ = 200

_INFO = plsc.get_sparse_core_info()
NC, NS = _INFO.num_cores, _INFO.num_subcores
NW = NC * NS
N_TOK = BATCH * SEQ
PER_W = N_TOK // NW
SEQ_PER_W = PER_W // SEQ
CH_SEQ = 2
CH = CH_SEQ * SEQ
NCHUNK = PER_W // CH
NGRP = CH // 16


def _sin_pe(seq_len, dim):
    pos = np.arange(seq_len, dtype=np.float32)[:, None]
    i = np.arange(0, dim, 2, dtype=np.float32)
    div = np.exp(-np.log(10000.0) * i / dim)
    pe = np.zeros((seq_len, dim), dtype=np.float32)
    pe[:, 0::2] = np.sin(pos * div)
    pe[:, 1::2] = np.cos(pos * div)
    return pe


_PE = _sin_pe(SEQ, DIM)

_MESH = plsc.VectorSubcoreMesh(core_axis_name="c", subcore_axis_name="s")


@functools.partial(
    pl.kernel,
    out_type=jax.ShapeDtypeStruct((BATCH, SEQ, DIM), jnp.float32),
    mesh=_MESH,
    scratch_types=[
        pltpu.VMEM((CH,), jnp.int32),
        pltpu.VMEM((CH,), jnp.int32),
        pltpu.VMEM((SEQ, DIM), jnp.float32),
        pltpu.VMEM((CH, DIM), jnp.float32),
        pltpu.VMEM((CH, DIM), jnp.float32),
        pltpu.SemaphoreType.DMA,
        pltpu.SemaphoreType.DMA,
        pltpu.SemaphoreType.DMA,
        pltpu.SemaphoreType.DMA,
    ],
)
def _embed_sc(tok_hbm, table_hbm, pe_hbm, out_hbm,
              idx0, idx1, pe_v, rows0, rows1, gsem0, gsem1, wsem0, wsem1):
    wid = lax.axis_index("s") * NC + lax.axis_index("c")
    base = wid * PER_W
    b0 = wid * SEQ_PER_W
    pltpu.sync_copy(pe_hbm, pe_v)

    idxs = (idx0, idx1)
    bufs = (rows0, rows1)
    gsems = (gsem0, gsem1)
    wsems = (wsem0, wsem1)

    def fire_gather(s, b):
        buf = bufs[b]
        sem = gsems[b]
        idx_v = idxs[b]
        pltpu.sync_copy(tok_hbm.at[pl.ds(base + s * CH, CH)], idx_v)

        def grp(k, c):
            tv = idx_v[pl.ds(k * 16, 16)]
            for j in range(16):
                t = tv[j]
                pltpu.async_copy(table_hbm.at[t], buf.at[k * 16 + j], sem)
            return c

        lax.fori_loop(0, NGRP, grp, 0)

    def wait_gather(b):
        pltpu.make_async_copy(
            table_hbm.at[pl.ds(0, CH)], bufs[b], gsems[b]
        ).wait()

    fire_gather(0, 0)
    writes = []
    for g in range(NCHUNK):
        b = g & 1
        if g + 1 < NCHUNK:
            if g >= 1:
                writes[2 * (g - 1)].wait()
                writes[2 * (g - 1) + 1].wait()
            fire_gather(g + 1, 1 - b)
        wait_gather(b)

        buf = bufs[b]

        @plsc.parallel_loop(0, SEQ, unroll=4)
        def _add(r):
            for k in range(CH_SEQ):
                for d in range(DIM // 16):
                    buf[k * SEQ + r, pl.ds(d * 16, 16)] = (
                        buf[k * SEQ + r, pl.ds(d * 16, 16)]
                        + pe_v[r, pl.ds(d * 16, 16)]
                    )

        for k in range(CH_SEQ):
            writes.append(pltpu.async_copy(
                buf.at[pl.ds(k * SEQ, SEQ)],
                out_hbm.at[b0 + g * CH_SEQ + k],
                wsems[b],
            ))
    for w in writes[-4:]:
        w.wait()


def kernel(tokens, table):
    tok_flat = tokens.astype(jnp.int32).reshape(-1)
    return _embed_sc(tok_flat, table, jnp.asarray(_PE))
